# baseline (speedup 1.0000x reference)
import jax
import jax.numpy as jnp
from jax import lax
from jax.experimental import pallas as pl
from jax.experimental.pallas import tpu as pltpu
from jax.experimental.pallas import tpu_sc as plsc

N = 10000
D = 128
E = 320000

NC = 2
NS = 16
L = 16
NW = NC * NS

NPAD = 10240
CH = 128
CHUNKS = 80
EPW = CH * CHUNKS
EP = EPW * NW
RPS = NPAD // NS

_mesh = plsc.VectorSubcoreMesh(core_axis_name="c", subcore_axis_name="s")


def _zero_tile(ref, rows, cols):
    z = jnp.zeros((L,), jnp.float32)
    for r in range(rows):
        for j in range(cols // L):
            ref[r, pl.ds(j * L, L)] = z


K = 8
SUP = CHUNKS // K
SUPE = K * CH


def _hist_body(dst_hbm, out_hbm, dstv, onesv, hbuf, ssem, acc):
    c = lax.axis_index("c")
    s = lax.axis_index("s")
    w = c * NS + s

    if True:
        one = jnp.full((L,), 1.0, jnp.float32)
        for j in range(CH // L):
            onesv[pl.ds(j * L, L)] = one
        z = jnp.zeros((L,), jnp.float32)
        for j in range(RPS // L):
            hbuf[pl.ds(j * L, L)] = z
        pltpu.sync_copy(hbuf, acc.at[pl.ds(s * RPS, RPS)])
        plsc.subcore_barrier()

        def sup(i, carry):
            row0 = w * CHUNKS + i * K
            pltpu.sync_copy(dst_hbm.at[pl.ds(row0, K)], dstv)
            for j in range(K):
                pltpu.async_copy(onesv, acc.at[dstv.at[j]], ssem, add=True)
            for j in range(K):
                pltpu.make_async_copy(onesv, acc.at[dstv.at[j]], ssem).wait()
            return carry

        lax.fori_loop(0, SUP, sup, 0)
        plsc.subcore_barrier()
        pltpu.sync_copy(acc.at[pl.ds(s * RPS, RPS)], hbuf)
        pltpu.sync_copy(hbuf, out_hbm.at[c, pl.ds(s * RPS, RPS)])


_hist = pl.kernel(
    _hist_body,
    out_type=jax.ShapeDtypeStruct((NC, NPAD), jnp.float32),
    mesh=_mesh,
    scratch_types=[
        pltpu.VMEM((K, CH), jnp.int32),
        pltpu.VMEM((CH,), jnp.float32),
        pltpu.VMEM((RPS,), jnp.float32),
        pltpu.SemaphoreType.DMA,
        pltpu.VMEM_SHARED((NPAD,), jnp.float32),
    ],
)


NB = 2


def _scatter_body(g_hbm, src_hbm, dst_hbm, out_hbm,
                  srca, srcb, dstf, rows0, rows1, zbuf,
                  gsem, ssem, psem, zsem, acc):
    c = lax.axis_index("c")
    s = lax.axis_index("s")
    w = c * NS + s
    bufs = (rows0, rows1)
    sbufs = (srca, srcb)

    def stage(b):
        return pltpu.make_async_copy(
            src_hbm.at[pl.ds(w * EPW + b * SUPE, SUPE)], sbufs[b % 2], psem)

    pltpu.async_copy(dst_hbm.at[pl.ds(w * CHUNKS, CHUNKS)], dstf, psem)
    stage(0).start()

    _zero_tile(zbuf, 16, D)

    for i in range(RPS // 16):
        pltpu.async_copy(zbuf, acc.at[pl.ds(s * RPS + i * 16, 16)], zsem)

    def gath(ci, buf):
        return pltpu.make_async_copy(
            g_hbm.at[sbufs[(ci // K) % 2].at[pl.ds((ci % K) * CH, CH)]],
            buf, gsem)

    def scat(ci, buf):
        return pltpu.make_async_copy(buf, acc.at[dstf.at[ci]], ssem)

    for i in range(RPS // 16):
        pltpu.make_async_copy(zbuf, acc.at[pl.ds(s * RPS + i * 16, 16)],
                              zsem).wait()
    pltpu.make_async_copy(dst_hbm.at[pl.ds(w * CHUNKS, CHUNKS)], dstf,
                          psem).wait()
    stage(0).wait()
    if SUP > 1:
        stage(1).start()
    plsc.subcore_barrier()
    gath(0, bufs[0]).start()

    for ci in range(CHUNKS):
        if ci + 1 < CHUNKS:
            if ci >= 1:
                scat(ci - 1, bufs[(ci - 1) % NB]).wait()
            if (ci + 1) % K == 0:
                b = (ci + 1) // K
                stage(b).wait()
                if b + 1 < SUP:
                    stage(b + 1).start()
            gath(ci + 1, bufs[(ci + 1) % NB]).start()
        gath(ci, bufs[ci % NB]).wait()
        pltpu.async_copy(bufs[ci % NB], acc.at[dstf.at[ci]], ssem, add=True)
    for ci in range(CHUNKS - NB, CHUNKS):
        scat(ci, bufs[ci % NB]).wait()
    plsc.subcore_barrier()

    WB = RPS // CH

    def whbm(jj):
        return out_hbm.at[c, pl.ds(s * RPS + jj * CH, CH)]

    for jj in range(WB):
        if jj >= 2:
            pltpu.make_async_copy(bufs[jj % 2], whbm(jj - 2), psem).wait()
        pltpu.sync_copy(acc.at[pl.ds(s * RPS + jj * CH, CH)], bufs[jj % 2])
        pltpu.async_copy(bufs[jj % 2], whbm(jj), psem)
    for jj in range(WB - 2, WB):
        pltpu.make_async_copy(bufs[jj % 2], whbm(jj), psem).wait()


_scatter = pl.kernel(
    _scatter_body,
    out_type=jax.ShapeDtypeStruct((NC, NPAD, D), jnp.float32),
    mesh=_mesh,
    scratch_types=[
        pltpu.VMEM((SUPE,), jnp.int32),
        pltpu.VMEM((SUPE,), jnp.int32),
        pltpu.VMEM((CHUNKS, CH), jnp.int32),
        pltpu.VMEM((CH, D), jnp.float32),
        pltpu.VMEM((CH, D), jnp.float32),
        pltpu.VMEM((16, D), jnp.float32),
        pltpu.SemaphoreType.DMA,
        pltpu.SemaphoreType.DMA,
        pltpu.SemaphoreType.DMA,
        pltpu.SemaphoreType.DMA,
        pltpu.VMEM_SHARED((NPAD, D), jnp.float32),
    ],
)


BLK = 2048
GRID = NPAD // BLK


def _tc0a_body(x_ref, w_ref, u_ref):
    u_ref[...] = jnp.dot(x_ref[...], w_ref[...],
                         preferred_element_type=jnp.float32)


def _tc0b_body(hist_ref, u_ref, dinv_ref, g_ref):
    hp = hist_ref[...]
    deg = (hp[0:1, :] + hp[1:2, :] + 1.0).reshape(BLK, 1)
    dinv = lax.rsqrt(deg)
    dinv_ref[...] = dinv
    g_ref[...] = dinv * u_ref[...]


def _tc1_body(s_ref, g_ref, dinv_ref, b_ref, w_ref, g1_ref):
    dinv = dinv_ref[...]
    h = dinv * (s_ref[0] + s_ref[1] + g_ref[...]) + b_ref[...]
    h = jnp.maximum(h, 0.0)
    g1_ref[...] = dinv * jnp.dot(h, w_ref[...], preferred_element_type=jnp.float32)


def _tc2_body(s_ref, g_ref, dinv_ref, b_ref, out_ref):
    out_ref[...] = (dinv_ref[...] * (s_ref[0] + s_ref[1] + g_ref[...])
                    + b_ref[...])


_col_spec = pl.BlockSpec((NC, BLK), lambda i: (0, i))
_row_spec = pl.BlockSpec((BLK, D), lambda i: (i, 0))
_dinv_spec = pl.BlockSpec((BLK, 1), lambda i: (i, 0))
_w_spec = pl.BlockSpec((D, D), lambda i: (0, 0))
_b_spec = pl.BlockSpec((1, D), lambda i: (0, 0))
_s_spec = pl.BlockSpec((NC, BLK, D), lambda i: (0, i, 0))

BLK2 = 2000
GRID2 = N // BLK2
_row_spec2 = pl.BlockSpec((BLK2, D), lambda i: (i, 0))
_dinv_spec2 = pl.BlockSpec((BLK2, 1), lambda i: (i, 0))
_s_spec2 = pl.BlockSpec((NC, BLK2, D), lambda i: (0, i, 0))

_tc0a = pl.pallas_call(
    _tc0a_body,
    grid=(GRID,),
    in_specs=[_row_spec, _w_spec],
    out_specs=_row_spec,
    out_shape=jax.ShapeDtypeStruct((NPAD, D), jnp.float32),
)

_tc0b = pl.pallas_call(
    _tc0b_body,
    grid=(GRID,),
    in_specs=[_col_spec, _row_spec],
    out_specs=[_dinv_spec, _row_spec],
    out_shape=[
        jax.ShapeDtypeStruct((NPAD, 1), jnp.float32),
        jax.ShapeDtypeStruct((NPAD, D), jnp.float32),
    ],
)

_tc1 = pl.pallas_call(
    _tc1_body,
    grid=(GRID,),
    in_specs=[_s_spec, _row_spec, _dinv_spec, _b_spec, _w_spec],
    out_specs=_row_spec,
    out_shape=jax.ShapeDtypeStruct((NPAD, D), jnp.float32),
)

_tc2 = pl.pallas_call(
    _tc2_body,
    grid=(GRID2,),
    in_specs=[_s_spec2, _row_spec2, _dinv_spec2, _b_spec],
    out_specs=_row_spec2,
    out_shape=jax.ShapeDtypeStruct((N, D), jnp.float32),
)


@jax.jit
def kernel(x, edge_index, W0, b0, W1, b1):
    src = edge_index[0].astype(jnp.int32)
    dst = edge_index[1].astype(jnp.int32)
    pad_idx = jnp.broadcast_to(
        jnp.arange(N, NPAD, dtype=jnp.int32), ((EP - E) // (NPAD - N), NPAD - N)
    ).reshape(-1)
    src_p = jnp.concatenate([src, pad_idx])
    dst_p = jnp.concatenate([dst, pad_idx])
    x_p = jnp.pad(x, ((0, NPAD - N), (0, 0)))
    b0r = b0.reshape(1, D)
    b1r = b1.reshape(1, D)

    dst2 = dst_p.reshape(EP // CH, CH)

    hist = _hist(dst2)
    u0 = _tc0a(x_p, W0)
    dinv, g0 = _tc0b(hist, u0)
    s0 = _scatter(g0, src_p, dst2)
    g1 = _tc1(s0, g0, dinv, b0r, W1)
    s1 = _scatter(g1, src_p, dst2)
    return _tc2(s1, g1, dinv, b1r)


# speedup vs baseline: 1.0051x; 1.0051x over previous
import jax
import jax.numpy as jnp
from jax import lax
from jax.experimental import pallas as pl
from jax.experimental.pallas import tpu as pltpu
from jax.experimental.pallas import tpu_sc as plsc

N = 10000
D = 128
E = 320000

NC = 2
NS = 16
L = 16
NW = NC * NS

NPAD = 10240
CH = 128
CHUNKS = 80
EPW = CH * CHUNKS
EP = EPW * NW
RPS = NPAD // NS

_mesh = plsc.VectorSubcoreMesh(core_axis_name="c", subcore_axis_name="s")


def _zero_tile(ref, rows, cols):
    z = jnp.zeros((L,), jnp.float32)
    for r in range(rows):
        for j in range(cols // L):
            ref[r, pl.ds(j * L, L)] = z


K = 8
SUP = CHUNKS // K
SUPE = K * CH


def _hist_body(dst_hbm, out_hbm, dstv, onesv, hbuf, ssem, acc):
    c = lax.axis_index("c")
    s = lax.axis_index("s")
    w = c * NS + s

    if True:
        one = jnp.full((L,), 1.0, jnp.float32)
        for j in range(CH // L):
            onesv[pl.ds(j * L, L)] = one
        z = jnp.zeros((L,), jnp.float32)
        for j in range(RPS // L):
            hbuf[pl.ds(j * L, L)] = z
        pltpu.sync_copy(hbuf, acc.at[pl.ds(s * RPS, RPS)])
        plsc.subcore_barrier()

        def sup(i, carry):
            row0 = w * CHUNKS + i * K
            pltpu.sync_copy(dst_hbm.at[pl.ds(row0, K)], dstv)
            for j in range(K):
                pltpu.async_copy(onesv, acc.at[dstv.at[j]], ssem, add=True)
            for j in range(K):
                pltpu.make_async_copy(onesv, acc.at[dstv.at[j]], ssem).wait()
            return carry

        lax.fori_loop(0, SUP, sup, 0)
        plsc.subcore_barrier()
        pltpu.sync_copy(acc.at[pl.ds(s * RPS, RPS)], hbuf)
        pltpu.sync_copy(hbuf, out_hbm.at[c, pl.ds(s * RPS, RPS)])


_hist = pl.kernel(
    _hist_body,
    out_type=jax.ShapeDtypeStruct((NC, NPAD), jnp.float32),
    mesh=_mesh,
    scratch_types=[
        pltpu.VMEM((K, CH), jnp.int32),
        pltpu.VMEM((CH,), jnp.float32),
        pltpu.VMEM((RPS,), jnp.float32),
        pltpu.SemaphoreType.DMA,
        pltpu.VMEM_SHARED((NPAD,), jnp.float32),
    ],
)


NB = 2


def _scatter_body(g_hbm, src_hbm, dst_hbm, out_hbm,
                  srca, srcb, dstf, rows0, rows1, zbuf,
                  gsem, ssem, psem, zsem, acc):
    c = lax.axis_index("c")
    s = lax.axis_index("s")
    w = c * NS + s
    bufs = (rows0, rows1)
    sbufs = (srca, srcb)

    def stage(b):
        return pltpu.make_async_copy(
            src_hbm.at[pl.ds(w * EPW + b * SUPE, SUPE)], sbufs[b % 2], psem)

    pltpu.async_copy(dst_hbm.at[pl.ds(w * CHUNKS, CHUNKS)], dstf, psem)
    stage(0).start()

    _zero_tile(zbuf, 16, D)

    for i in range(RPS // 16):
        pltpu.async_copy(zbuf, acc.at[pl.ds(s * RPS + i * 16, 16)], zsem)

    def gath(ci, buf):
        return pltpu.make_async_copy(
            g_hbm.at[sbufs[(ci // K) % 2].at[pl.ds((ci % K) * CH, CH)]],
            buf, gsem)

    def scat(ci, buf):
        return pltpu.make_async_copy(buf, acc.at[dstf.at[ci]], ssem)

    for i in range(RPS // 16):
        pltpu.make_async_copy(zbuf, acc.at[pl.ds(s * RPS + i * 16, 16)],
                              zsem).wait()
    pltpu.make_async_copy(dst_hbm.at[pl.ds(w * CHUNKS, CHUNKS)], dstf,
                          psem).wait()
    stage(0).wait()
    if SUP > 1:
        stage(1).start()
    plsc.subcore_barrier()
    gath(0, bufs[0]).start()

    for ci in range(CHUNKS):
        if ci + 1 < CHUNKS:
            if ci >= 1:
                scat(ci - 1, bufs[(ci - 1) % NB]).wait()
            if (ci + 1) % K == 0:
                b = (ci + 1) // K
                stage(b).wait()
                if b + 1 < SUP:
                    stage(b + 1).start()
            gath(ci + 1, bufs[(ci + 1) % NB]).start()
        gath(ci, bufs[ci % NB]).wait()
        pltpu.async_copy(bufs[ci % NB], acc.at[dstf.at[ci]], ssem, add=True)
    for ci in range(CHUNKS - NB, CHUNKS):
        scat(ci, bufs[ci % NB]).wait()
    plsc.subcore_barrier()

    WB = RPS // CH

    def whbm(jj):
        return out_hbm.at[c, pl.ds(s * RPS + jj * CH, CH)]

    for jj in range(WB):
        if jj >= 2:
            pltpu.make_async_copy(bufs[jj % 2], whbm(jj - 2), psem).wait()
        pltpu.sync_copy(acc.at[pl.ds(s * RPS + jj * CH, CH)], bufs[jj % 2])
        pltpu.async_copy(bufs[jj % 2], whbm(jj), psem)
    for jj in range(WB - 2, WB):
        pltpu.make_async_copy(bufs[jj % 2], whbm(jj), psem).wait()


_scatter = pl.kernel(
    _scatter_body,
    out_type=jax.ShapeDtypeStruct((NC, NPAD, D), jnp.float32),
    mesh=_mesh,
    scratch_types=[
        pltpu.VMEM((SUPE,), jnp.int32),
        pltpu.VMEM((SUPE,), jnp.int32),
        pltpu.VMEM((CHUNKS, CH), jnp.int32),
        pltpu.VMEM((CH, D), jnp.float32),
        pltpu.VMEM((CH, D), jnp.float32),
        pltpu.VMEM((16, D), jnp.float32),
        pltpu.SemaphoreType.DMA,
        pltpu.SemaphoreType.DMA,
        pltpu.SemaphoreType.DMA,
        pltpu.SemaphoreType.DMA,
        pltpu.VMEM_SHARED((NPAD, D), jnp.float32),
    ],
)


BLK = 2048
GRID = NPAD // BLK


def _tc0_body(hist_ref, x_ref, w_ref, dinv_ref, g_ref):
    hp = hist_ref[...]
    deg = (hp[0:1, :] + hp[1:2, :] + 1.0).reshape(BLK, 1)
    dinv = lax.rsqrt(deg)
    u = jnp.dot(x_ref[...], w_ref[...], preferred_element_type=jnp.float32)
    dinv_ref[...] = dinv
    g_ref[...] = dinv * u


def _tc1_body(s_ref, g_ref, dinv_ref, b_ref, w_ref, g1_ref):
    dinv = dinv_ref[...]
    h = dinv * (s_ref[0] + s_ref[1] + g_ref[...]) + b_ref[...]
    h = jnp.maximum(h, 0.0)
    g1_ref[...] = dinv * jnp.dot(h, w_ref[...], preferred_element_type=jnp.float32)


def _tc2_body(s_ref, g_ref, dinv_ref, b_ref, out_ref):
    out_ref[...] = (dinv_ref[...] * (s_ref[0] + s_ref[1] + g_ref[...])
                    + b_ref[...])


_col_spec = pl.BlockSpec((NC, BLK), lambda i: (0, i))
_row_spec = pl.BlockSpec((BLK, D), lambda i: (i, 0))
_dinv_spec = pl.BlockSpec((BLK, 1), lambda i: (i, 0))
_w_spec = pl.BlockSpec((D, D), lambda i: (0, 0))
_b_spec = pl.BlockSpec((1, D), lambda i: (0, 0))
_s_spec = pl.BlockSpec((NC, BLK, D), lambda i: (0, i, 0))

BLK2 = 2000
GRID2 = N // BLK2
_row_spec2 = pl.BlockSpec((BLK2, D), lambda i: (i, 0))
_dinv_spec2 = pl.BlockSpec((BLK2, 1), lambda i: (i, 0))
_s_spec2 = pl.BlockSpec((NC, BLK2, D), lambda i: (0, i, 0))

_tc0 = pl.pallas_call(
    _tc0_body,
    grid=(GRID,),
    in_specs=[_col_spec, _row_spec, _w_spec],
    out_specs=[_dinv_spec, _row_spec],
    out_shape=[
        jax.ShapeDtypeStruct((NPAD, 1), jnp.float32),
        jax.ShapeDtypeStruct((NPAD, D), jnp.float32),
    ],
)

_tc1 = pl.pallas_call(
    _tc1_body,
    grid=(GRID,),
    in_specs=[_s_spec, _row_spec, _dinv_spec, _b_spec, _w_spec],
    out_specs=_row_spec,
    out_shape=jax.ShapeDtypeStruct((NPAD, D), jnp.float32),
)

_tc2 = pl.pallas_call(
    _tc2_body,
    grid=(GRID2,),
    in_specs=[_s_spec2, _row_spec2, _dinv_spec2, _b_spec],
    out_specs=_row_spec2,
    out_shape=jax.ShapeDtypeStruct((N, D), jnp.float32),
)


@jax.jit
def kernel(x, edge_index, W0, b0, W1, b1):
    src = edge_index[0].astype(jnp.int32)
    dst = edge_index[1].astype(jnp.int32)
    pad_idx = jnp.broadcast_to(
        jnp.arange(N, NPAD, dtype=jnp.int32), ((EP - E) // (NPAD - N), NPAD - N)
    ).reshape(-1)
    src_p = jnp.concatenate([src, pad_idx])
    dst_p = jnp.concatenate([dst, pad_idx])
    x_p = jnp.pad(x, ((0, NPAD - N), (0, 0)))
    b0r = b0.reshape(1, D)
    b1r = b1.reshape(1, D)

    dst2 = dst_p.reshape(EP // CH, CH)

    hist = _hist(dst2)
    dinv, g0 = _tc0(hist, x_p, W0)
    s0 = _scatter(g0, src_p, dst2)
    g1 = _tc1(s0, g0, dinv, b0r, W1)
    s1 = _scatter(g1, src_p, dst2)
    return _tc2(s1, g1, dinv, b1r)
